# baseline (speedup 1.0000x reference)
import functools
import math

import jax
import jax.numpy as jnp
from jax import lax
from jax.experimental import pallas as pl
from jax.experimental.pallas import tpu as pltpu
from jax.experimental.pallas import tpu_sc as plsc

B = 2048
D_IN = 512
H = 1024
E = 16
NC = 20
TOPK = 4
EPS = 1e-5
BT = 256
NB = B // BT
NPAIR = B * TOPK
TRG = 128
NT = NPAIR // TRG + E
NPAD = NT * TRG
CAP = B
NW = 32
ZCH = NPAD // E

_BN_SCALE = 1.0 / math.sqrt(1.0 + EPS)
_INV_SQRT2 = 1.0 / math.sqrt(2.0)

_SC_MESH = plsc.VectorSubcoreMesh(core_axis_name="c", subcore_axis_name="s")


def _gelu(x):
    return x * 0.5 * (1.0 + jax.lax.erf(x * _INV_SQRT2))


def _gate_body(wifi_ref, rfid_ref, wg1_ref, bg1_ref, lng_ref, lnb_ref,
               wg2_ref, bg2_ref,
               comb_ref, tki_ref, tkw_ref, offs_ref, te_ref, cnt_ref):
    b = pl.program_id(0)

    @pl.when(b == 0)
    def _init():
        cnt_ref[...] = jnp.zeros((1, E), jnp.int32)

    x = jnp.concatenate([wifi_ref[...], rfid_ref[...]], axis=1)
    comb_ref[...] = x.astype(jnp.bfloat16)
    g1 = jax.lax.dot_general(x, wg1_ref[...], (((1,), (1,)), ((), ())),
                             preferred_element_type=jnp.float32)
    g1 = g1 + bg1_ref[...]
    m = jnp.mean(g1, axis=1, keepdims=True)
    v = jnp.mean((g1 - m) ** 2, axis=1, keepdims=True)
    g1 = (g1 - m) * jax.lax.rsqrt(v + EPS) * lng_ref[...] + lnb_ref[...]
    g1 = _gelu(g1)
    logits = jax.lax.dot_general(g1, wg2_ref[...], (((1,), (1,)), ((), ())),
                                 preferred_element_type=jnp.float32)
    logits = logits + bg2_ref[...]
    logits = logits - jnp.max(logits, axis=1, keepdims=True)
    eg = jnp.exp(logits)
    gate = eg / jnp.sum(eg, axis=1, keepdims=True)

    eidx = jax.lax.broadcasted_iota(jnp.int32, (BT, E), 1)
    work = gate
    vals = []
    idxs = []
    for _ in range(TOPK):
        mx = jnp.max(work, axis=1, keepdims=True)
        amx = jnp.argmax(work, axis=1).astype(jnp.int32)[:, None]
        vals.append(mx)
        idxs.append(amx)
        work = jnp.where(eidx == amx, -jnp.inf, work)
    v0 = vals[0]
    exps = [jnp.exp(vv - v0) for vv in vals]
    tot = exps[0]
    for ecur in exps[1:]:
        tot = tot + ecur
    tki_ref[...] = jnp.concatenate(idxs, axis=1)
    tkw_ref[...] = jnp.concatenate([ee / tot for ee in exps], axis=1)

    c = jnp.zeros((1, E), jnp.int32)
    for amx in idxs:
        c = c + jnp.sum((eidx == amx).astype(jnp.int32), axis=0,
                        keepdims=True)
    cnt_ref[...] += c

    @pl.when(b == NB - 1)
    def _finish():
        cnt = cnt_ref[...]
        padded = ((cnt + (TRG - 1)) // TRG) * TRG
        pf = padded.astype(jnp.float32)
        tri = (jax.lax.broadcasted_iota(jnp.int32, (E, E), 0) <
               jax.lax.broadcasted_iota(jnp.int32, (E, E), 1)
               ).astype(jnp.float32)
        offs = jax.lax.dot_general(pf, tri, (((1,), (0,)), ((), ())),
                                   preferred_element_type=jnp.float32)
        offs_i = offs.astype(jnp.int32)
        offs_ref[...] = offs_i
        tstart = jax.lax.broadcasted_iota(jnp.int32, (NT, E), 0) * TRG
        cmp = (jnp.broadcast_to(offs_i, (NT, E)) <= tstart).astype(jnp.int32)
        te = jnp.sum(cmp, axis=1) - 1
        nv = (offs_i[0, E - 1] + padded[0, E - 1]) // TRG
        te_ref[...] = jnp.concatenate(
            [te, nv.reshape(1)]).reshape(1, NT + 1)


@functools.partial(
    pl.kernel,
    out_type=(jax.ShapeDtypeStruct((NPAD + E,), jnp.int32),
              jax.ShapeDtypeStruct((NPAD + E,), jnp.float32),
              jax.ShapeDtypeStruct((NPAIR + E,), jnp.int32)),
    mesh=_SC_MESH,
    compiler_params=pltpu.CompilerParams(needs_layout_passes=False),
    scratch_types=[
        pltpu.VMEM((NPAIR,), jnp.int32),
        pltpu.VMEM((NPAIR,), jnp.float32),
        pltpu.VMEM((E,), jnp.int32),
        pltpu.VMEM((CAP + 16,), jnp.int32),
        pltpu.VMEM((CAP + 16,), jnp.int32),
        pltpu.VMEM((CAP + 16,), jnp.float32),
        pltpu.VMEM((CAP + 16,), jnp.int32),
        pltpu.VMEM((CAP + 16,), jnp.int32),
        pltpu.VMEM((ZCH,), jnp.int32),
        pltpu.VMEM((ZCH,), jnp.float32),
        pltpu.VMEM_SHARED((NPAD + E,), jnp.int32),
        pltpu.VMEM_SHARED((NPAD + E,), jnp.float32),
        pltpu.VMEM_SHARED((NPAIR + E,), jnp.int32),
        pltpu.SemaphoreType.DMA,
    ])
def _route(tki_hbm, tkw_hbm, offs_hbm, st_hbm, sg_hbm, pp_hbm,
           idx_v, gate_v, offs_v, sc_dst, sc_tok, sc_gate, sc_ppi, sc_ppv,
           zi_v, zf_v, sh_st, sh_sg, sh_pp, sem):
    c = lax.axis_index("c")
    s = lax.axis_index("s")

    @pl.when(c == 0)
    def _core0():
        z16i = jnp.zeros((16,), jnp.int32)
        z16f = jnp.zeros((16,), jnp.float32)
        for i in range(ZCH // 16):
            zi_v[pl.ds(i * 16, 16)] = z16i
            zf_v[pl.ds(i * 16, 16)] = z16f
        pltpu.sync_copy(zi_v, sh_st.at[pl.ds(s * ZCH, ZCH)])
        pltpu.sync_copy(zf_v, sh_sg.at[pl.ds(s * ZCH, ZCH)])
        dump_st = jnp.full((16,), NPAD + s, jnp.int32)
        dump_pp = jnp.full((16,), NPAIR + s, jnp.int32)
        for i in range((CAP + 16) // 16):
            sl = pl.ds(i * 16, 16)
            sc_dst[sl] = dump_st
            sc_ppi[sl] = dump_pp
            sc_tok[sl] = z16i
            sc_gate[sl] = z16f
            sc_ppv[sl] = z16i
        plsc.subcore_barrier()

        pltpu.sync_copy(tki_hbm, idx_v)
        pltpu.sync_copy(tkw_hbm, gate_v)
        pltpu.sync_copy(offs_hbm, offs_v)
        lanes = lax.iota(jnp.int32, 16)
        ovec = offs_v[...]
        obase = jnp.max(jnp.where(lanes == s, ovec, jnp.int32(0)))

        def step(j, cur):
            v = idx_v[pl.ds(j * 16, 16)]
            g = gate_v[pl.ds(j * 16, 16)]
            msk = v == s
            cm = plsc.cumsum(jnp.where(msk, 1, 0).astype(jnp.int32))
            dest = obase + cur - 1 + cm
            pid = j * 16 + lanes
            tok = lax.shift_right_logical(pid, 2)
            csl = pl.ds(cur, 16)
            plsc.store_compressed(sc_dst.at[csl], dest, mask=msk)
            plsc.store_compressed(sc_tok.at[csl], tok, mask=msk)
            plsc.store_compressed(sc_gate.at[csl], g, mask=msk)
            plsc.store_compressed(sc_ppi.at[csl], pid, mask=msk)
            plsc.store_compressed(sc_ppv.at[csl], dest, mask=msk)
            return cur + jnp.max(cm)

        lax.fori_loop(0, NPAIR // 16, step, jnp.int32(0))
        pltpu.async_copy(sc_tok, sh_st.at[sc_dst], sem).wait()
        pltpu.async_copy(sc_gate, sh_sg.at[sc_dst], sem).wait()
        pltpu.async_copy(sc_ppv, sh_pp.at[sc_ppi], sem).wait()
        plsc.subcore_barrier()
        pltpu.sync_copy(sh_st.at[pl.ds(s * ZCH, ZCH)],
                        st_hbm.at[pl.ds(s * ZCH, ZCH)])
        pltpu.sync_copy(sh_sg.at[pl.ds(s * ZCH, ZCH)],
                        sg_hbm.at[pl.ds(s * ZCH, ZCH)])
        pcs = NPAIR // E
        pltpu.sync_copy(sh_pp.at[pl.ds(s * pcs, pcs)],
                        pp_hbm.at[pl.ds(s * pcs, pcs)])


def _group_body(te_ref, st_ref, comb_ref, we1_ref, be1_ref, bn1g_ref,
                bn1b_ref, we2_ref, be2_ref, bn2g_ref, bn2b_ref, sg_ref,
                y_ref, w1b_ref, w2b_ref, xacc_ref):
    t = pl.program_id(0)
    prev = te_ref[jnp.maximum(t - 1, 0)]

    @pl.when((t == 0) | (te_ref[t] != prev))
    def _cast():
        w1b_ref[...] = we1_ref[0].astype(jnp.bfloat16)
        w2b_ref[...] = we2_ref[0].astype(jnp.bfloat16)

    nv = te_ref[NT]
    valid = t < nv

    stc = st_ref[...]
    real = sg_ref[...] > 0.0
    lo = jnp.min(jnp.where(real, stc, B - 1))
    hi = jnp.max(jnp.where(real, stc, 0))

    @pl.when(valid)
    def _zero():
        xacc_ref[...] = jnp.zeros((TRG, D_IN), jnp.float32)

    kb = 256
    for k0 in range(0, B, kb):
        @pl.when(valid & (lo < k0 + kb) & (hi >= k0))
        def _chunk(k0=k0):
            cols = jax.lax.broadcasted_iota(jnp.int32, (TRG, kb), 1) + k0
            oh = (cols == stc).astype(jnp.bfloat16)
            xacc_ref[...] += jax.lax.dot_general(
                oh, comb_ref[pl.ds(k0, kb), :], (((1,), (0,)), ((), ())),
                preferred_element_type=jnp.float32)

    @pl.when(valid)
    def _ffn():
        x = xacc_ref[...].astype(jnp.bfloat16)
        h1 = jax.lax.dot_general(x, w1b_ref[...], (((1,), (1,)), ((), ())),
                                 preferred_element_type=jnp.float32)
        h1 = h1 + be1_ref[0]
        h1 = _gelu(h1 * _BN_SCALE * bn1g_ref[0] + bn1b_ref[0])
        h2 = jax.lax.dot_general(h1.astype(jnp.bfloat16), w2b_ref[...],
                                 (((1,), (1,)), ((), ())),
                                 preferred_element_type=jnp.float32)
        h2 = h2 + be2_ref[0]
        h2 = _gelu(h2 * _BN_SCALE * bn2g_ref[0] + bn2b_ref[0])
        y_ref[...] = h2 * sg_ref[...]


@functools.partial(
    pl.kernel,
    out_type=jax.ShapeDtypeStruct((B, H), jnp.float32),
    mesh=_SC_MESH,
    compiler_params=pltpu.CompilerParams(needs_layout_passes=False),
    scratch_types=[
        pltpu.VMEM((B * TOPK // NW,), jnp.int32),
        pltpu.VMEM((32, H), jnp.float32),
        pltpu.VMEM((32, H), jnp.float32),
        pltpu.VMEM((8, H), jnp.float32),
        pltpu.SemaphoreType.DMA,
        pltpu.SemaphoreType.DMA,
    ])
def _combine(y_hbm, pp_hbm, fused_hbm, ppv, yv0, yv1, ov, sem0, sem1):
    wid = lax.axis_index("s") * 2 + lax.axis_index("c")
    tpw = B // NW
    nch = tpw // 8
    tbase = wid * tpw
    pltpu.sync_copy(pp_hbm.at[pl.ds(tbase * TOPK, tpw * TOPK)], ppv)
    bufs = (yv0, yv1)
    sems = (sem0, sem1)
    cps = [None, None]
    cps[0] = pltpu.async_copy(y_hbm.at[ppv.at[pl.ds(0, 32)]], yv0, sem0)
    for ch in range(nch):
        cur = bufs[ch % 2]
        if ch + 1 < nch:
            j = (ch + 1) % 2
            cps[j] = pltpu.async_copy(
                y_hbm.at[ppv.at[pl.ds((ch + 1) * 32, 32)]], bufs[j], sems[j])
        cps[ch % 2].wait()
        for t in range(8):
            def hstep(hi, _, t=t, cur=cur):
                for u in range(4):
                    sl = pl.ds((hi * 4 + u) * 16, 16)
                    ov[t, sl] = (cur[4 * t, sl] + cur[4 * t + 1, sl] +
                                 cur[4 * t + 2, sl] + cur[4 * t + 3, sl])
                return 0
            lax.fori_loop(0, H // (16 * 4), hstep, 0)
        pltpu.sync_copy(ov, fused_hbm.at[pl.ds(tbase + ch * 8, 8)])


def _final_body(fused_ref, wf1_ref, bf1_ref, bnfg_ref, bnfb_ref, wf2_ref,
                bf2_ref, out_ref):
    f1 = jax.lax.dot_general(fused_ref[...], wf1_ref[...],
                             (((1,), (1,)), ((), ())),
                             preferred_element_type=jnp.float32)
    f1 = f1 + bf1_ref[...]
    f1 = _gelu(f1 * _BN_SCALE * bnfg_ref[...] + bnfb_ref[...])
    out = jax.lax.dot_general(f1, wf2_ref[...], (((1,), (1,)), ((), ())),
                              preferred_element_type=jnp.float32)
    out_ref[...] = out + bf2_ref[...]


def kernel(wifi_feat, rfid_feat, Wg1, bg1, ln_g, ln_b, Wg2, bg2,
           We1, be1, bn1_g, bn1_b, We2, be2, bn2_g, bn2_b,
           Wf1, bf1, bnf_g, bnf_b, Wf2, bf2):
    comb, tki, tkw, offs, te = pl.pallas_call(
        _gate_body,
        grid=(NB,),
        in_specs=[
            pl.BlockSpec((BT, 256), lambda b: (b, 0)),
            pl.BlockSpec((BT, 256), lambda b: (b, 0)),
            pl.BlockSpec((H, D_IN), lambda b: (0, 0)),
            pl.BlockSpec((1, H), lambda b: (0, 0)),
            pl.BlockSpec((1, H), lambda b: (0, 0)),
            pl.BlockSpec((1, H), lambda b: (0, 0)),
            pl.BlockSpec((E, H), lambda b: (0, 0)),
            pl.BlockSpec((1, E), lambda b: (0, 0)),
        ],
        out_specs=[
            pl.BlockSpec((BT, D_IN), lambda b: (b, 0)),
            pl.BlockSpec((BT, TOPK), lambda b: (b, 0)),
            pl.BlockSpec((BT, TOPK), lambda b: (b, 0)),
            pl.BlockSpec((1, E), lambda b: (0, 0)),
            pl.BlockSpec((1, NT + 1), lambda b: (0, 0)),
        ],
        out_shape=[
            jax.ShapeDtypeStruct((B, D_IN), jnp.bfloat16),
            jax.ShapeDtypeStruct((B, TOPK), jnp.int32),
            jax.ShapeDtypeStruct((B, TOPK), jnp.float32),
            jax.ShapeDtypeStruct((1, E), jnp.int32),
            jax.ShapeDtypeStruct((1, NT + 1), jnp.int32),
        ],
        scratch_shapes=[pltpu.VMEM((1, E), jnp.int32)],
    )(wifi_feat, rfid_feat, Wg1, bg1.reshape(1, H), ln_g.reshape(1, H),
      ln_b.reshape(1, H), Wg2, bg2.reshape(1, E))

    st, sg, pp = _route(tki.reshape(NPAIR), tkw.reshape(NPAIR),
                        offs.reshape(E))

    y = pl.pallas_call(
        _group_body,
        grid_spec=pltpu.PrefetchScalarGridSpec(
            num_scalar_prefetch=1,
            grid=(NT,),
            in_specs=[
                pl.BlockSpec((TRG, 1), lambda t, te: (t, 0)),
                pl.BlockSpec((B, D_IN), lambda t, te: (0, 0)),
                pl.BlockSpec((1, H, D_IN), lambda t, te: (te[t], 0, 0)),
                pl.BlockSpec((1, 1, H), lambda t, te: (te[t], 0, 0)),
                pl.BlockSpec((1, 1, H), lambda t, te: (te[t], 0, 0)),
                pl.BlockSpec((1, 1, H), lambda t, te: (te[t], 0, 0)),
                pl.BlockSpec((1, H, H), lambda t, te: (te[t], 0, 0)),
                pl.BlockSpec((1, 1, H), lambda t, te: (te[t], 0, 0)),
                pl.BlockSpec((1, 1, H), lambda t, te: (te[t], 0, 0)),
                pl.BlockSpec((1, 1, H), lambda t, te: (te[t], 0, 0)),
                pl.BlockSpec((TRG, 1), lambda t, te: (t, 0)),
            ],
            out_specs=pl.BlockSpec((TRG, H), lambda t, te: (t, 0)),
            scratch_shapes=[
                pltpu.VMEM((H, D_IN), jnp.bfloat16),
                pltpu.VMEM((H, H), jnp.bfloat16),
                pltpu.VMEM((TRG, D_IN), jnp.float32),
            ],
        ),
        out_shape=jax.ShapeDtypeStruct((NPAD, H), jnp.float32),
    )(te.reshape(NT + 1), st[:NPAD].reshape(NPAD, 1), comb, We1,
      be1.reshape(E, 1, H),
      bn1_g.reshape(E, 1, H), bn1_b.reshape(E, 1, H), We2,
      be2.reshape(E, 1, H), bn2_g.reshape(E, 1, H), bn2_b.reshape(E, 1, H),
      sg[:NPAD].reshape(NPAD, 1))

    fused = _combine(y, pp)

    out = pl.pallas_call(
        _final_body,
        grid=(NB,),
        in_specs=[
            pl.BlockSpec((BT, H), lambda b: (b, 0)),
            pl.BlockSpec((512, H), lambda b: (0, 0)),
            pl.BlockSpec((1, 512), lambda b: (0, 0)),
            pl.BlockSpec((1, 512), lambda b: (0, 0)),
            pl.BlockSpec((1, 512), lambda b: (0, 0)),
            pl.BlockSpec((NC, 512), lambda b: (0, 0)),
            pl.BlockSpec((1, NC), lambda b: (0, 0)),
        ],
        out_specs=pl.BlockSpec((BT, NC), lambda b: (b, 0)),
        out_shape=jax.ShapeDtypeStruct((B, NC), jnp.float32),
    )(fused, Wf1, bf1.reshape(1, 512), bnf_g.reshape(1, 512),
      bnf_b.reshape(1, 512), Wf2, bf2.reshape(1, NC))
    return out


# speedup vs baseline: 1.0823x; 1.0823x over previous
import functools
import math

import jax
import jax.numpy as jnp
from jax import lax
from jax.experimental import pallas as pl
from jax.experimental.pallas import tpu as pltpu
from jax.experimental.pallas import tpu_sc as plsc

B = 2048
D_IN = 512
H = 1024
E = 16
NC = 20
TOPK = 4
EPS = 1e-5
BT = 256
NB = B // BT
NPAIR = B * TOPK
TRG = 128
NT = NPAIR // TRG + E
NPAD = NT * TRG
CAP = B
NW = 32
ZCH = NPAD // E

_BN_SCALE = 1.0 / math.sqrt(1.0 + EPS)
_INV_SQRT2 = 1.0 / math.sqrt(2.0)

_SC_MESH = plsc.VectorSubcoreMesh(core_axis_name="c", subcore_axis_name="s")


def _gelu(x):
    return x * 0.5 * (1.0 + jax.lax.erf(x * _INV_SQRT2))


def _gate_body(wifi_ref, rfid_ref, wg1_ref, bg1_ref, lng_ref, lnb_ref,
               wg2_ref, bg2_ref,
               comb_ref, tki_ref, tkw_ref, offs_ref, te_ref, cnt_ref):
    b = pl.program_id(0)

    @pl.when(b == 0)
    def _init():
        cnt_ref[...] = jnp.zeros((1, E), jnp.int32)

    x = jnp.concatenate([wifi_ref[...], rfid_ref[...]], axis=1)
    comb_ref[...] = x.astype(jnp.bfloat16)
    g1 = jax.lax.dot_general(x, wg1_ref[...], (((1,), (1,)), ((), ())),
                             preferred_element_type=jnp.float32)
    g1 = g1 + bg1_ref[...]
    m = jnp.mean(g1, axis=1, keepdims=True)
    v = jnp.mean((g1 - m) ** 2, axis=1, keepdims=True)
    g1 = (g1 - m) * jax.lax.rsqrt(v + EPS) * lng_ref[...] + lnb_ref[...]
    g1 = _gelu(g1)
    logits = jax.lax.dot_general(g1, wg2_ref[...], (((1,), (1,)), ((), ())),
                                 preferred_element_type=jnp.float32)
    logits = logits + bg2_ref[...]
    logits = logits - jnp.max(logits, axis=1, keepdims=True)
    eg = jnp.exp(logits)
    gate = eg / jnp.sum(eg, axis=1, keepdims=True)

    eidx = jax.lax.broadcasted_iota(jnp.int32, (BT, E), 1)
    work = gate
    vals = []
    idxs = []
    for _ in range(TOPK):
        mx = jnp.max(work, axis=1, keepdims=True)
        amx = jnp.argmax(work, axis=1).astype(jnp.int32)[:, None]
        vals.append(mx)
        idxs.append(amx)
        work = jnp.where(eidx == amx, -jnp.inf, work)
    v0 = vals[0]
    exps = [jnp.exp(vv - v0) for vv in vals]
    tot = exps[0]
    for ecur in exps[1:]:
        tot = tot + ecur
    tki_ref[...] = jnp.concatenate(idxs, axis=1)
    tkw_ref[...] = jnp.concatenate([ee / tot for ee in exps], axis=1)

    c = jnp.zeros((1, E), jnp.int32)
    for amx in idxs:
        c = c + jnp.sum((eidx == amx).astype(jnp.int32), axis=0,
                        keepdims=True)
    cnt_ref[...] += c

    @pl.when(b == NB - 1)
    def _finish():
        cnt = cnt_ref[...]
        padded = ((cnt + (TRG - 1)) // TRG) * TRG
        pf = padded.astype(jnp.float32)
        tri = (jax.lax.broadcasted_iota(jnp.int32, (E, E), 0) <
               jax.lax.broadcasted_iota(jnp.int32, (E, E), 1)
               ).astype(jnp.float32)
        offs = jax.lax.dot_general(pf, tri, (((1,), (0,)), ((), ())),
                                   preferred_element_type=jnp.float32)
        offs_i = offs.astype(jnp.int32)
        offs_ref[...] = offs_i
        tstart = jax.lax.broadcasted_iota(jnp.int32, (NT, E), 0) * TRG
        cmp = (jnp.broadcast_to(offs_i, (NT, E)) <= tstart).astype(jnp.int32)
        te = jnp.sum(cmp, axis=1) - 1
        nv = (offs_i[0, E - 1] + padded[0, E - 1]) // TRG
        te_ref[...] = jnp.concatenate(
            [te, nv.reshape(1)]).reshape(1, NT + 1)


@functools.partial(
    pl.kernel,
    out_type=(jax.ShapeDtypeStruct((NPAD + E,), jnp.int32),
              jax.ShapeDtypeStruct((NPAD + E,), jnp.float32),
              jax.ShapeDtypeStruct((NPAIR + E,), jnp.int32)),
    mesh=_SC_MESH,
    compiler_params=pltpu.CompilerParams(needs_layout_passes=False),
    scratch_types=[
        pltpu.VMEM((NPAIR,), jnp.int32),
        pltpu.VMEM((NPAIR,), jnp.float32),
        pltpu.VMEM((E,), jnp.int32),
        pltpu.VMEM((CAP + 16,), jnp.int32),
        pltpu.VMEM((CAP + 16,), jnp.int32),
        pltpu.VMEM((CAP + 16,), jnp.float32),
        pltpu.VMEM((CAP + 16,), jnp.int32),
        pltpu.VMEM((CAP + 16,), jnp.int32),
        pltpu.VMEM((ZCH,), jnp.int32),
        pltpu.VMEM((ZCH,), jnp.float32),
        pltpu.VMEM_SHARED((NPAD + E,), jnp.int32),
        pltpu.VMEM_SHARED((NPAD + E,), jnp.float32),
        pltpu.VMEM_SHARED((NPAIR + E,), jnp.int32),
        pltpu.SemaphoreType.DMA,
    ])
def _route(tki_hbm, tkw_hbm, offs_hbm, st_hbm, sg_hbm, pp_hbm,
           idx_v, gate_v, offs_v, sc_dst, sc_tok, sc_gate, sc_ppi, sc_ppv,
           zi_v, zf_v, sh_st, sh_sg, sh_pp, sem):
    c = lax.axis_index("c")
    s = lax.axis_index("s")

    @pl.when(c == 0)
    def _core0():
        z16i = jnp.zeros((16,), jnp.int32)
        z16f = jnp.zeros((16,), jnp.float32)
        for i in range(ZCH // 16):
            zi_v[pl.ds(i * 16, 16)] = z16i
            zf_v[pl.ds(i * 16, 16)] = z16f
        pltpu.sync_copy(zi_v, sh_st.at[pl.ds(s * ZCH, ZCH)])
        pltpu.sync_copy(zf_v, sh_sg.at[pl.ds(s * ZCH, ZCH)])
        dump_st = jnp.full((16,), NPAD + s, jnp.int32)
        dump_pp = jnp.full((16,), NPAIR + s, jnp.int32)
        for i in range((CAP + 16) // 16):
            sl = pl.ds(i * 16, 16)
            sc_dst[sl] = dump_st
            sc_ppi[sl] = dump_pp
            sc_tok[sl] = z16i
            sc_gate[sl] = z16f
            sc_ppv[sl] = z16i
        plsc.subcore_barrier()

        pltpu.sync_copy(tki_hbm, idx_v)
        pltpu.sync_copy(tkw_hbm, gate_v)
        pltpu.sync_copy(offs_hbm, offs_v)
        lanes = lax.iota(jnp.int32, 16)
        ovec = offs_v[...]
        obase = jnp.max(jnp.where(lanes == s, ovec, jnp.int32(0)))

        def step(j, cur):
            v = idx_v[pl.ds(j * 16, 16)]
            g = gate_v[pl.ds(j * 16, 16)]
            msk = v == s
            cm = plsc.cumsum(jnp.where(msk, 1, 0).astype(jnp.int32))
            dest = obase + cur - 1 + cm
            pid = j * 16 + lanes
            tok = lax.shift_right_logical(pid, 2)
            csl = pl.ds(cur, 16)
            plsc.store_compressed(sc_dst.at[csl], dest, mask=msk)
            plsc.store_compressed(sc_tok.at[csl], tok, mask=msk)
            plsc.store_compressed(sc_gate.at[csl], g, mask=msk)
            plsc.store_compressed(sc_ppi.at[csl], pid, mask=msk)
            plsc.store_compressed(sc_ppv.at[csl], dest, mask=msk)
            return cur + jnp.max(cm)

        lax.fori_loop(0, NPAIR // 16, step, jnp.int32(0))
        pltpu.async_copy(sc_tok, sh_st.at[sc_dst], sem).wait()
        pltpu.async_copy(sc_gate, sh_sg.at[sc_dst], sem).wait()
        pltpu.async_copy(sc_ppv, sh_pp.at[sc_ppi], sem).wait()
        plsc.subcore_barrier()
        pltpu.sync_copy(sh_st.at[pl.ds(s * ZCH, ZCH)],
                        st_hbm.at[pl.ds(s * ZCH, ZCH)])
        pltpu.sync_copy(sh_sg.at[pl.ds(s * ZCH, ZCH)],
                        sg_hbm.at[pl.ds(s * ZCH, ZCH)])
        pcs = NPAIR // E
        pltpu.sync_copy(sh_pp.at[pl.ds(s * pcs, pcs)],
                        pp_hbm.at[pl.ds(s * pcs, pcs)])


def _group_body(te_ref, st_ref, comb_ref, we1_ref, be1_ref, bn1g_ref,
                bn1b_ref, we2_ref, be2_ref, bn2g_ref, bn2b_ref, sg_ref,
                y_ref, w1b_ref, w2b_ref, xacc_ref):
    t = pl.program_id(0)
    prev = te_ref[jnp.maximum(t - 1, 0)]

    @pl.when((t == 0) | (te_ref[t] != prev))
    def _cast():
        w1b_ref[...] = we1_ref[0].astype(jnp.bfloat16)
        w2b_ref[...] = we2_ref[0].astype(jnp.bfloat16)

    nv = te_ref[NT]

    @pl.when(t < nv)
    def _work():
        stc = st_ref[...]
        xacc = jnp.zeros((TRG, D_IN), jnp.float32)
        kb = 256
        for k0 in range(0, B, kb):
            cols = jax.lax.broadcasted_iota(jnp.int32, (TRG, kb), 1) + k0
            oh = (cols == stc).astype(jnp.bfloat16)
            xacc = xacc + jax.lax.dot_general(
                oh, comb_ref[pl.ds(k0, kb), :], (((1,), (0,)), ((), ())),
                preferred_element_type=jnp.float32)
        x = xacc.astype(jnp.bfloat16)
        h1 = jax.lax.dot_general(x, w1b_ref[...], (((1,), (1,)), ((), ())),
                                 preferred_element_type=jnp.float32)
        h1 = h1 + be1_ref[0]
        h1 = _gelu(h1 * _BN_SCALE * bn1g_ref[0] + bn1b_ref[0])
        h2 = jax.lax.dot_general(h1.astype(jnp.bfloat16), w2b_ref[...],
                                 (((1,), (1,)), ((), ())),
                                 preferred_element_type=jnp.float32)
        h2 = h2 + be2_ref[0]
        h2 = _gelu(h2 * _BN_SCALE * bn2g_ref[0] + bn2b_ref[0])
        y_ref[...] = h2 * sg_ref[...]


@functools.partial(
    pl.kernel,
    out_type=jax.ShapeDtypeStruct((B, H), jnp.float32),
    mesh=_SC_MESH,
    compiler_params=pltpu.CompilerParams(needs_layout_passes=False),
    scratch_types=[
        pltpu.VMEM((B * TOPK // NW,), jnp.int32),
        pltpu.VMEM((32, H), jnp.float32),
        pltpu.VMEM((32, H), jnp.float32),
        pltpu.VMEM((8, H), jnp.float32),
        pltpu.SemaphoreType.DMA,
        pltpu.SemaphoreType.DMA,
    ])
def _combine(y_hbm, pp_hbm, fused_hbm, ppv, yv0, yv1, ov, sem0, sem1):
    wid = lax.axis_index("s") * 2 + lax.axis_index("c")
    tpw = B // NW
    nch = tpw // 8
    tbase = wid * tpw
    pltpu.sync_copy(pp_hbm.at[pl.ds(tbase * TOPK, tpw * TOPK)], ppv)
    bufs = (yv0, yv1)
    sems = (sem0, sem1)
    cps = [None, None]
    cps[0] = pltpu.async_copy(y_hbm.at[ppv.at[pl.ds(0, 32)]], yv0, sem0)
    for ch in range(nch):
        cur = bufs[ch % 2]
        if ch + 1 < nch:
            j = (ch + 1) % 2
            cps[j] = pltpu.async_copy(
                y_hbm.at[ppv.at[pl.ds((ch + 1) * 32, 32)]], bufs[j], sems[j])
        cps[ch % 2].wait()
        for t in range(8):
            def hstep(hi, _, t=t, cur=cur):
                for u in range(4):
                    sl = pl.ds((hi * 4 + u) * 16, 16)
                    ov[t, sl] = (cur[4 * t, sl] + cur[4 * t + 1, sl] +
                                 cur[4 * t + 2, sl] + cur[4 * t + 3, sl])
                return 0
            lax.fori_loop(0, H // (16 * 4), hstep, 0)
        pltpu.sync_copy(ov, fused_hbm.at[pl.ds(tbase + ch * 8, 8)])


def _final_body(fused_ref, wf1_ref, bf1_ref, bnfg_ref, bnfb_ref, wf2_ref,
                bf2_ref, out_ref):
    f1 = jax.lax.dot_general(fused_ref[...], wf1_ref[...],
                             (((1,), (1,)), ((), ())),
                             preferred_element_type=jnp.float32)
    f1 = f1 + bf1_ref[...]
    f1 = _gelu(f1 * _BN_SCALE * bnfg_ref[...] + bnfb_ref[...])
    out = jax.lax.dot_general(f1, wf2_ref[...], (((1,), (1,)), ((), ())),
                              preferred_element_type=jnp.float32)
    out_ref[...] = out + bf2_ref[...]


def kernel(wifi_feat, rfid_feat, Wg1, bg1, ln_g, ln_b, Wg2, bg2,
           We1, be1, bn1_g, bn1_b, We2, be2, bn2_g, bn2_b,
           Wf1, bf1, bnf_g, bnf_b, Wf2, bf2):
    comb, tki, tkw, offs, te = pl.pallas_call(
        _gate_body,
        grid=(NB,),
        in_specs=[
            pl.BlockSpec((BT, 256), lambda b: (b, 0)),
            pl.BlockSpec((BT, 256), lambda b: (b, 0)),
            pl.BlockSpec((H, D_IN), lambda b: (0, 0)),
            pl.BlockSpec((1, H), lambda b: (0, 0)),
            pl.BlockSpec((1, H), lambda b: (0, 0)),
            pl.BlockSpec((1, H), lambda b: (0, 0)),
            pl.BlockSpec((E, H), lambda b: (0, 0)),
            pl.BlockSpec((1, E), lambda b: (0, 0)),
        ],
        out_specs=[
            pl.BlockSpec((BT, D_IN), lambda b: (b, 0)),
            pl.BlockSpec((BT, TOPK), lambda b: (b, 0)),
            pl.BlockSpec((BT, TOPK), lambda b: (b, 0)),
            pl.BlockSpec((1, E), lambda b: (0, 0)),
            pl.BlockSpec((1, NT + 1), lambda b: (0, 0)),
        ],
        out_shape=[
            jax.ShapeDtypeStruct((B, D_IN), jnp.bfloat16),
            jax.ShapeDtypeStruct((B, TOPK), jnp.int32),
            jax.ShapeDtypeStruct((B, TOPK), jnp.float32),
            jax.ShapeDtypeStruct((1, E), jnp.int32),
            jax.ShapeDtypeStruct((1, NT + 1), jnp.int32),
        ],
        scratch_shapes=[pltpu.VMEM((1, E), jnp.int32)],
    )(wifi_feat, rfid_feat, Wg1, bg1.reshape(1, H), ln_g.reshape(1, H),
      ln_b.reshape(1, H), Wg2, bg2.reshape(1, E))

    st, sg, pp = _route(tki.reshape(NPAIR), tkw.reshape(NPAIR),
                        offs.reshape(E))

    y = pl.pallas_call(
        _group_body,
        grid_spec=pltpu.PrefetchScalarGridSpec(
            num_scalar_prefetch=1,
            grid=(NT,),
            in_specs=[
                pl.BlockSpec((TRG, 1), lambda t, te: (t, 0)),
                pl.BlockSpec((B, D_IN), lambda t, te: (0, 0)),
                pl.BlockSpec((1, H, D_IN), lambda t, te: (te[t], 0, 0)),
                pl.BlockSpec((1, 1, H), lambda t, te: (te[t], 0, 0)),
                pl.BlockSpec((1, 1, H), lambda t, te: (te[t], 0, 0)),
                pl.BlockSpec((1, 1, H), lambda t, te: (te[t], 0, 0)),
                pl.BlockSpec((1, H, H), lambda t, te: (te[t], 0, 0)),
                pl.BlockSpec((1, 1, H), lambda t, te: (te[t], 0, 0)),
                pl.BlockSpec((1, 1, H), lambda t, te: (te[t], 0, 0)),
                pl.BlockSpec((1, 1, H), lambda t, te: (te[t], 0, 0)),
                pl.BlockSpec((TRG, 1), lambda t, te: (t, 0)),
            ],
            out_specs=pl.BlockSpec((TRG, H), lambda t, te: (t, 0)),
            scratch_shapes=[
                pltpu.VMEM((H, D_IN), jnp.bfloat16),
                pltpu.VMEM((H, H), jnp.bfloat16),
                pltpu.VMEM((TRG, D_IN), jnp.float32),
            ],
        ),
        out_shape=jax.ShapeDtypeStruct((NPAD, H), jnp.float32),
    )(te.reshape(NT + 1), st[:NPAD].reshape(NPAD, 1), comb, We1,
      be1.reshape(E, 1, H),
      bn1_g.reshape(E, 1, H), bn1_b.reshape(E, 1, H), We2,
      be2.reshape(E, 1, H), bn2_g.reshape(E, 1, H), bn2_b.reshape(E, 1, H),
      sg[:NPAD].reshape(NPAD, 1))

    fused = _combine(y, pp)

    out = pl.pallas_call(
        _final_body,
        grid=(NB,),
        in_specs=[
            pl.BlockSpec((BT, H), lambda b: (b, 0)),
            pl.BlockSpec((512, H), lambda b: (0, 0)),
            pl.BlockSpec((1, 512), lambda b: (0, 0)),
            pl.BlockSpec((1, 512), lambda b: (0, 0)),
            pl.BlockSpec((1, 512), lambda b: (0, 0)),
            pl.BlockSpec((NC, 512), lambda b: (0, 0)),
            pl.BlockSpec((1, NC), lambda b: (0, 0)),
        ],
        out_specs=pl.BlockSpec((BT, NC), lambda b: (b, 0)),
        out_shape=jax.ShapeDtypeStruct((B, NC), jnp.float32),
    )(fused, Wf1, bf1.reshape(1, 512), bnf_g.reshape(1, 512),
      bnf_b.reshape(1, 512), Wf2, bf2.reshape(1, NC))
    return out
